# Initial kernel scaffold; baseline (speedup 1.0000x reference)
import jax
import jax.numpy as jnp
from jax.experimental import pallas as pl


def kernel(x, tables):
    raise NotImplementedError("write your pallas kernel here")


# speedup vs baseline: 1.1201x; 1.1201x over previous
import functools

import jax
import jax.numpy as jnp
from jax import lax
from jax.experimental import pallas as pl
from jax.experimental.pallas import tpu as pltpu
from jax.experimental.pallas import tpu_sc as plsc

_NC = 2
_NS = 16
_L = 16
_NW = _NC * _NS


@functools.lru_cache(maxsize=None)
def _build(T, V, D, B):
    N = B * T
    n_per_w = N // _NW
    CH = 1664
    NCH = n_per_w // CH
    assert n_per_w % CH == 0 and n_per_w % _L == 0

    mesh = plsc.VectorSubcoreMesh(
        core_axis_name="c", subcore_axis_name="s",
        num_cores=_NC, num_subcores=_NS)

    @functools.partial(
        pl.kernel,
        out_type=jax.ShapeDtypeStruct((N, D), jnp.float32),
        mesh=mesh,
        scratch_types=[
            pltpu.VMEM((n_per_w,), jnp.int32),
            pltpu.VMEM((CH, D), jnp.float32),
            pltpu.SemaphoreType.DMA,
        ],
        compiler_params=pltpu.CompilerParams(use_tc_tiling_on_sc=False),
    )
    def k(x_hbm, tab_hbm, out_hbm, idx_v, rows_v, sem):
        wid = lax.axis_index("s") * _NC + lax.axis_index("c")
        base = wid * n_per_w
        pltpu.sync_copy(x_hbm.at[pl.ds(base, n_per_w)], idx_v)

        def body(j, carry):
            off = j * _L
            g = lax.iota(jnp.int32, _L) + (base + off)
            t = lax.rem(g, T)
            idx_v[pl.ds(off, _L)] = idx_v[pl.ds(off, _L)] + t * V
            return carry
        lax.fori_loop(0, n_per_w // _L, body, 0)

        for c in range(NCH):
            pltpu.async_copy(
                tab_hbm.at[idx_v.at[pl.ds(c * CH, CH)]], rows_v, sem).wait()
            pltpu.sync_copy(rows_v, out_hbm.at[pl.ds(base + c * CH, CH)])

    return k


def kernel(x, tables):
    if x.ndim <= 1:
        x = x[None, :]
    B, T = x.shape
    _, V, D = tables.shape
    out = _build(T, V, D, B)(x.reshape(B * T), tables.reshape(T * V, D))
    return out.reshape(B, T * D)


# speedup vs baseline: 1.1221x; 1.0018x over previous
import functools

import jax
import jax.numpy as jnp
from jax import lax
from jax.experimental import pallas as pl
from jax.experimental.pallas import tpu as pltpu
from jax.experimental.pallas import tpu_sc as plsc

_NC = 2
_NS = 16
_L = 16
_NW = _NC * _NS
_NBUF = 3


@functools.lru_cache(maxsize=None)
def _build(T, V, D, B):
    N = B * T
    n_per_w = N // _NW
    CH = 1024
    NCH = n_per_w // CH
    assert n_per_w % CH == 0 and n_per_w % _L == 0 and NCH >= _NBUF

    mesh = plsc.VectorSubcoreMesh(
        core_axis_name="c", subcore_axis_name="s",
        num_cores=_NC, num_subcores=_NS)

    @functools.partial(
        pl.kernel,
        out_type=jax.ShapeDtypeStruct((N, D), jnp.float32),
        mesh=mesh,
        scratch_types=[
            pltpu.VMEM((n_per_w,), jnp.int32),
        ] + [pltpu.VMEM((CH, D), jnp.float32) for _ in range(_NBUF)]
          + [pltpu.SemaphoreType.DMA for _ in range(2 * _NBUF)],
        compiler_params=pltpu.CompilerParams(use_tc_tiling_on_sc=False),
    )
    def k(x_hbm, tab_hbm, out_hbm, idx_v, *bufs_sems):
        bufs = bufs_sems[:_NBUF]
        gsems = bufs_sems[_NBUF:2 * _NBUF]
        wsems = bufs_sems[2 * _NBUF:]
        wid = lax.axis_index("s") * _NC + lax.axis_index("c")
        base = wid * n_per_w
        pltpu.sync_copy(x_hbm.at[pl.ds(base, n_per_w)], idx_v)

        lanes = lax.iota(jnp.int32, _L)
        off0 = ((base + lanes) % T) * V
        step = (_L % T) * V
        wrap = T * V

        def body(j, off):
            pos = j * _L
            idx_v[pl.ds(pos, _L)] = idx_v[pl.ds(pos, _L)] + off
            nxt = off + step
            return jnp.where(nxt >= wrap, nxt - wrap, nxt)
        lax.fori_loop(0, n_per_w // _L, body, off0)

        def gather(c):
            return pltpu.async_copy(
                tab_hbm.at[idx_v.at[pl.ds(c * CH, CH)]],
                bufs[c % _NBUF], gsems[c % _NBUF])

        def writeback(c):
            return pltpu.async_copy(
                bufs[c % _NBUF], out_hbm.at[pl.ds(base + c * CH, CH)],
                wsems[c % _NBUF])

        gcopies = [None] * NCH
        wcopies = [None] * NCH
        for c in range(_NBUF):
            gcopies[c] = gather(c)
        for c in range(NCH):
            gcopies[c].wait()
            wcopies[c] = writeback(c)
            if c + _NBUF < NCH:
                wcopies[c].wait()
                gcopies[c + _NBUF] = gather(c + _NBUF)
        for c in range(NCH - _NBUF, NCH):
            if wcopies[c] is not None and c + _NBUF >= NCH:
                wcopies[c].wait()

    return k


def kernel(x, tables):
    if x.ndim <= 1:
        x = x[None, :]
    B, T = x.shape
    _, V, D = tables.shape
    out = _build(T, V, D, B)(x.reshape(B * T), tables.reshape(T * V, D))
    return out.reshape(B, T * D)
